# baseline (speedup 1.0000x reference)
import functools

import jax
import jax.numpy as jnp
from jax import lax
from jax.experimental import pallas as pl
from jax.experimental.pallas import tpu as pltpu
from jax.experimental.pallas import tpu_sc as plsc

_NC = 2
_NS = 16
_NW = _NC * _NS
_CH = 128

_F32 = jnp.float32


def _pack2(t, d):
    hi = lax.bitcast_convert_type(t[:, :d], jnp.int32)
    lo = lax.bitcast_convert_type(t[:, d:], jnp.int32)
    hi = (hi + 0x8000) & jnp.int32(-65536)
    lo = lax.shift_right_logical(lo + 0x8000, 16)
    return hi | lo


def _unpack_hi(g):
    return lax.bitcast_convert_type(g & jnp.int32(-65536), _F32)


def _unpack_lo(g):
    return lax.bitcast_convert_type(lax.shift_left(g, 16), _F32)


def _proj_body(x_ref, wd_ref, ws_ref, od_ref, os_ref):
    xb = x_ref[...]
    d = x_ref.shape[1]
    od_ref[...] = _pack2(
        jnp.dot(xb, wd_ref[...], preferred_element_type=_F32), d)
    os_ref[...] = _pack2(
        jnp.dot(xb, ws_ref[...], preferred_element_type=_F32), d)


def _project(x, wd, ws, bn, nrows):
    n, d = x.shape
    dd = wd.shape[1]
    return pl.pallas_call(
        _proj_body,
        grid=(n // bn,),
        in_specs=[
            pl.BlockSpec((bn, d), lambda i: (i, 0)),
            pl.BlockSpec((d, dd), lambda i: (0, 0)),
            pl.BlockSpec((d, dd), lambda i: (0, 0)),
        ],
        out_specs=[
            pl.BlockSpec((bn, d), lambda i: (i, 0)),
            pl.BlockSpec((bn, d), lambda i: (i, 0)),
        ],
        out_shape=[jax.ShapeDtypeStruct((nrows, d), jnp.int32)] * 2,
    )(x, wd, ws)


def _bdot(a, w):
    return jnp.dot(a.astype(jnp.bfloat16), w.astype(jnp.bfloat16),
                   preferred_element_type=_F32)


def _edge_body(gd_ref, gs_ref, ea_ref, ew1c_ref, eb1_ref, ew2_ref, eb2_ref,
               mw1c_ref, mb1_ref, mw2_ref, mb2_ref, ne_ref, msg_ref):
    gd = gd_ref[...]
    gs = gs_ref[...]
    he = (_unpack_hi(gd) + _unpack_hi(gs) + _bdot(ea_ref[...], ew1c_ref[...])
          + eb1_ref[...])
    he = jnp.maximum(he, 0.0)
    ne = _bdot(he, ew2_ref[...]) + eb2_ref[...]
    hm = (_unpack_lo(gd) + _unpack_lo(gs) + _bdot(ne, mw1c_ref[...])
          + mb1_ref[...])
    hm = jnp.maximum(hm, 0.0)
    msg_ref[...] = _bdot(hm, mw2_ref[...]) + mb2_ref[...]
    ne_ref[...] = ne.astype(ne_ref.dtype)


def _edge_mlps(gd, gs, ea, ew1c, eb1, ew2, eb2, mw1c, mb1, mw2, mb2, be,
               ne_dtype):
    ep, d = gd.shape
    ew = ea.shape[1]
    wspec = lambda sh: pl.BlockSpec(sh, lambda i: (0, 0))
    return pl.pallas_call(
        _edge_body,
        grid=(ep // be,),
        in_specs=[
            pl.BlockSpec((be, d), lambda i: (i, 0)),
            pl.BlockSpec((be, d), lambda i: (i, 0)),
            pl.BlockSpec((be, ew), lambda i: (i, 0)),
            wspec((ew, d)), wspec((1, d)), wspec((d, d)), wspec((1, d)),
            wspec((d, d)), wspec((1, d)), wspec((d, d)), wspec((1, d)),
        ],
        out_specs=[
            pl.BlockSpec((be, d), lambda i: (i, 0)),
            pl.BlockSpec((be, d), lambda i: (i, 0)),
        ],
        out_shape=[jax.ShapeDtypeStruct((ep, d), ne_dtype),
                   jax.ShapeDtypeStruct((ep, d), _F32)],
    )(gd, gs, ea, ew1c, eb1, ew2, eb2, mw1c, mb1, mw2, mb2)


def _node_body(x_ref, p0_ref, p1_ref, p2_ref, p3_ref, w1a_ref, w1b_ref,
               b1_ref, w2_ref, b2_ref, o_ref):
    aggr = p0_ref[0] + p1_ref[0] + p2_ref[0] + p3_ref[0]
    h = (jnp.dot(x_ref[...], w1a_ref[...], preferred_element_type=_F32)
         + jnp.dot(aggr, w1b_ref[...], preferred_element_type=_F32)
         + b1_ref[...])
    h = jnp.maximum(h, 0.0)
    o_ref[...] = jnp.dot(h, w2_ref[...], preferred_element_type=_F32) + b2_ref[...]


def _node_mlp(x, parts_a, parts_b, w1a, w1b, b1, w2, b2, bn):
    n, d = x.shape
    wspec = lambda sh: pl.BlockSpec(sh, lambda i: (0, 0))
    pspec = lambda c: pl.BlockSpec((1, bn, d), lambda i, c=c: (c, i, 0))
    return pl.pallas_call(
        _node_body,
        grid=(n // bn,),
        in_specs=[
            pl.BlockSpec((bn, d), lambda i: (i, 0)),
            pspec(0), pspec(1), pspec(0), pspec(1),
            wspec((d, d)), wspec((d, d)), wspec((1, d)), wspec((d, d)),
            wspec((1, d)),
        ],
        out_specs=pl.BlockSpec((bn, d), lambda i: (i, 0)),
        out_shape=jax.ShapeDtypeStruct((n, d), _F32),
    )(x, parts_a, parts_a, parts_b, parts_b, w1a, w1b, b1, w2, b2)


_NB = 2
_NBS = 2
_CHS = 112


def _gather_two(table_d, table_s, idx_d, idx_s):
    ep = idx_d.shape[0]
    nrows, d = table_d.shape
    per_w = ep // _NS
    nch = per_w // _CH
    ngr = nch // _NB
    rs = nrows // _NS
    mesh = plsc.VectorSubcoreMesh(core_axis_name="c", subcore_axis_name="s")

    @functools.partial(
        pl.kernel,
        out_type=[jax.ShapeDtypeStruct((ep, d), jnp.int32)] * 2,
        mesh=mesh,
        scratch_types=(
            [pltpu.VMEM((per_w,), jnp.int32)]
            + [pltpu.VMEM((_CH, d), jnp.int32)] * _NB
            + [pltpu.SemaphoreType.DMA] * (2 * _NB)
            + [pltpu.VMEM_SHARED((nrows, d), jnp.int32)]
        ),
    )
    def gather_kernel(td_hbm, ts_hbm, id_hbm, is_hbm, gd_hbm, gs_hbm,
                      idx_all, b0, b1, g0, g1, w0, w1, tstage):
        cid = lax.axis_index("c")
        sid = lax.axis_index("s")
        bufs = (b0, b1)
        gsem = (g0, g1)
        wsem = (w0, w1)
        base = sid * per_w

        def run(table, idh, outh):
            pltpu.sync_copy(table.at[pl.ds(sid * rs, rs)],
                            tstage.at[pl.ds(sid * rs, rs)])
            pltpu.sync_copy(idh.at[pl.ds(base, per_w)], idx_all)
            plsc.subcore_barrier()

            def grp(g, carry):
                for b in range(_NB):
                    j = g * _NB + b

                    @pl.when(g > 0)
                    def _():
                        pltpu.make_async_copy(
                            outh.at[pl.ds(0, _CH)], bufs[b], wsem[b]).wait()

                    pltpu.async_copy(
                        tstage.at[idx_all.at[pl.ds(j * _CH, _CH)]],
                        bufs[b], gsem[b])
                for b in range(_NB):
                    j = g * _NB + b
                    pltpu.make_async_copy(
                        outh.at[pl.ds(0, _CH)], bufs[b], gsem[b]).wait()
                    pltpu.async_copy(
                        bufs[b], outh.at[pl.ds(base + j * _CH, _CH)], wsem[b])
                return carry

            lax.fori_loop(0, ngr, grp, 0)
            for b in range(_NB):
                pltpu.make_async_copy(
                    outh.at[pl.ds(0, _CH)], bufs[b], wsem[b]).wait()

        @pl.when(cid == 0)
        def _():
            run(td_hbm, id_hbm, gd_hbm)

        @pl.when(cid == 1)
        def _():
            run(ts_hbm, is_hbm, gs_hbm)

    return gather_kernel(table_d, table_s, idx_d, idx_s)


def _scatter_add(msg, idx, zeros, n):
    ep, d = msg.shape
    npad = zeros.shape[0]
    per_w = ep // _NW
    nch = per_w // _CHS
    ngr = nch // _NBS
    rpt = npad // _NS
    mesh = plsc.VectorSubcoreMesh(core_axis_name="c", subcore_axis_name="s")

    @functools.partial(
        pl.kernel,
        out_type=jax.ShapeDtypeStruct((2, npad, d), _F32),
        mesh=mesh,
        scratch_types=(
            [pltpu.VMEM((nch, _CHS), jnp.int32)]
            + [pltpu.VMEM((_CHS, d), _F32)] * _NBS
            + [pltpu.SemaphoreType.DMA] * (2 * _NBS)
            + [pltpu.VMEM_SHARED((npad, d), _F32)]
        ),
    )
    def scatter_kernel(msg_hbm, idx_hbm, z_hbm, out_hbm,
                       idx_all, b0, b1, l0, l1, a0, a1, acc):
        cid = lax.axis_index("c")
        sid = lax.axis_index("s")
        wid = sid * _NC + cid
        bufs = (b0, b1)
        lsem = (l0, l1)
        asem = (a0, a1)

        pltpu.sync_copy(z_hbm.at[pl.ds(sid * rpt, rpt)],
                        acc.at[pl.ds(sid * rpt, rpt)])
        pltpu.sync_copy(idx_hbm.at[wid], idx_all)
        plsc.subcore_barrier()

        def grp(g, carry):
            for b in range(_NBS):
                j = g * _NBS + b

                @pl.when(g > 0)
                def _():
                    pltpu.make_async_copy(
                        msg_hbm.at[pl.ds(0, _CHS)], bufs[b], asem[b]).wait()

                pltpu.async_copy(
                    msg_hbm.at[pl.ds(wid * per_w + j * _CHS, _CHS)],
                    bufs[b], lsem[b])
            for b in range(_NBS):
                j = g * _NBS + b
                pltpu.make_async_copy(
                    msg_hbm.at[pl.ds(0, _CHS)], bufs[b], lsem[b]).wait()
                pltpu.async_copy(bufs[b], acc.at[idx_all.at[j]], asem[b],
                                 add=True)
            return carry

        lax.fori_loop(0, ngr, grp, 0)
        for b in range(_NBS):
            pltpu.make_async_copy(
                msg_hbm.at[pl.ds(0, _CHS)], bufs[b], asem[b]).wait()
        plsc.subcore_barrier()
        pltpu.sync_copy(acc.at[pl.ds(sid * rpt, rpt)],
                        out_hbm.at[cid, pl.ds(sid * rpt, rpt)])

    return scatter_kernel(msg, idx, zeros)


def kernel(x, edge_index, edge_attr,
           l0_ew1, l0_eb1, l0_ew2, l0_eb2,
           l0_mw1, l0_mb1, l0_mw2, l0_mb2,
           l0_nw1, l0_nb1, l0_nw2, l0_nb2,
           l1_ew1, l1_eb1, l1_ew2, l1_eb2,
           l1_mw1, l1_mb1, l1_mw2, l1_mb2,
           l1_nw1, l1_nb1, l1_nw2, l1_nb2):
    n, d = x.shape
    e = edge_index.shape[1]
    ed = edge_attr.shape[1]
    e_real = e + n
    quant = 2 * _NW * _CH
    ep = ((e_real + quant - 1) // quant) * quant
    eph = ep // 2
    pad = ep - e_real
    npad = ((n + 1 + 8 * _NS - 1) // (8 * _NS)) * (8 * _NS)

    loop = jnp.arange(n, dtype=jnp.int32)
    zpad_i = jnp.zeros((pad,), jnp.int32)
    src = jnp.concatenate([edge_index[0].astype(jnp.int32), loop, zpad_i])
    dst_g = jnp.concatenate([edge_index[1].astype(jnp.int32), loop, zpad_i])
    dst_s = jnp.concatenate([edge_index[1].astype(jnp.int32), loop,
                             jnp.full((pad,), n, jnp.int32)])
    ea = jnp.concatenate(
        [edge_attr, jnp.zeros((n + pad, ed), edge_attr.dtype)], axis=0)
    zeros_acc = jnp.zeros((npad, d), _F32)

    srcs = [src[:eph], src[eph:]]
    dstg = [dst_g[:eph], dst_g[eph:]]
    dsts = [dst_s[:eph].reshape(_NW, -1, _CHS),
            dst_s[eph:].reshape(_NW, -1, _CHS)]
    eas = [ea[:eph], ea[eph:]]

    layers = [
        (l0_ew1, l0_eb1, l0_ew2, l0_eb2, l0_mw1, l0_mb1, l0_mw2, l0_mb2,
         l0_nw1, l0_nb1, l0_nw2, l0_nb2),
        (l1_ew1, l1_eb1, l1_ew2, l1_eb2, l1_mw1, l1_mb1, l1_mw2, l1_mb2,
         l1_nw1, l1_nb1, l1_nw2, l1_nb2),
    ]

    bn = 1000
    be = 1024

    for li, (ew1, eb1, ew2, eb2, mw1, mb1, mw2, mb2,
             nw1, nb1, nw2, nb2) in enumerate(layers):
        wd = jnp.concatenate([ew1[:d], mw1[:d]], axis=1)
        wsc = jnp.concatenate([ew1[d:2 * d], mw1[d:2 * d]], axis=1)
        ne_dtype = jnp.bfloat16 if li == 0 else _F32
        td, ts = _project(x, wd, wsc, bn, npad)
        nes = []
        parts = []
        for h in (0, 1):
            gd, gs = _gather_two(td, ts, dstg[h], srcs[h])
            ne, msg = _edge_mlps(
                gd, gs, eas[h],
                ew1[2 * d:], eb1.reshape(1, d), ew2, eb2.reshape(1, d),
                mw1[2 * d:], mb1.reshape(1, d), mw2, mb2.reshape(1, d), be,
                ne_dtype)
            parts.append(_scatter_add(msg, dsts[h], zeros_acc, n))
            nes.append(ne)
        x = _node_mlp(x, parts[0], parts[1], nw1[:d], nw1[d:],
                      nb1.reshape(1, d), nw2, nb2.reshape(1, d), bn)
        eas = nes

    return (x, jnp.concatenate(eas)[:e_real])


# speedup vs baseline: 1.1371x; 1.1371x over previous
import functools

import jax
import jax.numpy as jnp
from jax import lax
from jax.experimental import pallas as pl
from jax.experimental.pallas import tpu as pltpu
from jax.experimental.pallas import tpu_sc as plsc

_NC = 2
_NS = 16
_NW = _NC * _NS
_CH = 128

_F32 = jnp.float32


def _pack2(t, d):
    hi = lax.bitcast_convert_type(t[:, :d], jnp.int32)
    lo = lax.bitcast_convert_type(t[:, d:], jnp.int32)
    hi = (hi + 0x8000) & jnp.int32(-65536)
    lo = lax.shift_right_logical(lo + 0x8000, 16)
    return hi | lo


def _unpack_hi(g):
    return lax.bitcast_convert_type(g & jnp.int32(-65536), _F32)


def _unpack_lo(g):
    return lax.bitcast_convert_type(lax.shift_left(g, 16), _F32)


def _proj_body(x_ref, wd_ref, ws_ref, od_ref, os_ref):
    xb = x_ref[...]
    d = x_ref.shape[1]
    od_ref[...] = _pack2(
        jnp.dot(xb, wd_ref[...], preferred_element_type=_F32), d)
    os_ref[...] = _pack2(
        jnp.dot(xb, ws_ref[...], preferred_element_type=_F32), d)


def _project(x, wd, ws, bn, nrows):
    n, d = x.shape
    dd = wd.shape[1]
    return pl.pallas_call(
        _proj_body,
        grid=(n // bn,),
        in_specs=[
            pl.BlockSpec((bn, d), lambda i: (i, 0)),
            pl.BlockSpec((d, dd), lambda i: (0, 0)),
            pl.BlockSpec((d, dd), lambda i: (0, 0)),
        ],
        out_specs=[
            pl.BlockSpec((bn, d), lambda i: (i, 0)),
            pl.BlockSpec((bn, d), lambda i: (i, 0)),
        ],
        out_shape=[jax.ShapeDtypeStruct((nrows, d), jnp.int32)] * 2,
    )(x, wd, ws)


def _bdot(a, w):
    return jnp.dot(a.astype(jnp.bfloat16), w.astype(jnp.bfloat16),
                   preferred_element_type=_F32)


def _edge_body(gd_ref, gs_ref, ea_ref, ew1c_ref, eb1_ref, ew2_ref, eb2_ref,
               mw1c_ref, mb1_ref, mw2_ref, mb2_ref, ne_ref, msg_ref):
    gd = gd_ref[...]
    gs = gs_ref[...]
    he = (_unpack_hi(gd) + _unpack_hi(gs) + _bdot(ea_ref[...], ew1c_ref[...])
          + eb1_ref[...])
    he = jnp.maximum(he, 0.0)
    ne = _bdot(he, ew2_ref[...]) + eb2_ref[...]
    hm = (_unpack_lo(gd) + _unpack_lo(gs) + _bdot(ne, mw1c_ref[...])
          + mb1_ref[...])
    hm = jnp.maximum(hm, 0.0)
    msg_ref[...] = _bdot(hm, mw2_ref[...]) + mb2_ref[...]
    ne_ref[...] = ne.astype(ne_ref.dtype)


def _edge_mlps(gd, gs, ea, ew1c, eb1, ew2, eb2, mw1c, mb1, mw2, mb2, be,
               ne_dtype):
    ep, d = gd.shape
    ew = ea.shape[1]
    wspec = lambda sh: pl.BlockSpec(sh, lambda i: (0, 0))
    return pl.pallas_call(
        _edge_body,
        grid=(ep // be,),
        in_specs=[
            pl.BlockSpec((be, d), lambda i: (i, 0)),
            pl.BlockSpec((be, d), lambda i: (i, 0)),
            pl.BlockSpec((be, ew), lambda i: (i, 0)),
            wspec((ew, d)), wspec((1, d)), wspec((d, d)), wspec((1, d)),
            wspec((d, d)), wspec((1, d)), wspec((d, d)), wspec((1, d)),
        ],
        out_specs=[
            pl.BlockSpec((be, d), lambda i: (i, 0)),
            pl.BlockSpec((be, d), lambda i: (i, 0)),
        ],
        out_shape=[jax.ShapeDtypeStruct((ep, d), ne_dtype),
                   jax.ShapeDtypeStruct((ep, d), _F32)],
    )(gd, gs, ea, ew1c, eb1, ew2, eb2, mw1c, mb1, mw2, mb2)


def _node_body(x_ref, p0_ref, p1_ref, p2_ref, p3_ref, w1a_ref, w1b_ref,
               b1_ref, w2_ref, b2_ref, o_ref):
    aggr = p0_ref[0] + p1_ref[0] + p2_ref[0] + p3_ref[0]
    h = (jnp.dot(x_ref[...], w1a_ref[...], preferred_element_type=_F32)
         + jnp.dot(aggr, w1b_ref[...], preferred_element_type=_F32)
         + b1_ref[...])
    h = jnp.maximum(h, 0.0)
    o_ref[...] = jnp.dot(h, w2_ref[...], preferred_element_type=_F32) + b2_ref[...]


def _node_mlp(x, parts_a, parts_b, w1a, w1b, b1, w2, b2, bn):
    n, d = x.shape
    wspec = lambda sh: pl.BlockSpec(sh, lambda i: (0, 0))
    pspec = lambda c: pl.BlockSpec((1, bn, d), lambda i, c=c: (c, i, 0))
    return pl.pallas_call(
        _node_body,
        grid=(n // bn,),
        in_specs=[
            pl.BlockSpec((bn, d), lambda i: (i, 0)),
            pspec(0), pspec(1), pspec(0), pspec(1),
            wspec((d, d)), wspec((d, d)), wspec((1, d)), wspec((d, d)),
            wspec((1, d)),
        ],
        out_specs=pl.BlockSpec((bn, d), lambda i: (i, 0)),
        out_shape=jax.ShapeDtypeStruct((n, d), _F32),
    )(x, parts_a, parts_a, parts_b, parts_b, w1a, w1b, b1, w2, b2)


_NB = 2
_NBS = 2
_CHS = 96


def _gather_two(table_d, table_s, idx_d, idx_s):
    ep = idx_d.shape[0]
    nrows, d = table_d.shape
    per_w = ep // _NS
    nch = per_w // _CH
    ngr = nch // _NB
    rs = nrows // _NS
    mesh = plsc.VectorSubcoreMesh(core_axis_name="c", subcore_axis_name="s")

    @functools.partial(
        pl.kernel,
        out_type=[jax.ShapeDtypeStruct((ep, d), jnp.int32)] * 2,
        mesh=mesh,
        scratch_types=(
            [pltpu.VMEM((per_w,), jnp.int32)]
            + [pltpu.VMEM((_CH, d), jnp.int32)] * _NB
            + [pltpu.SemaphoreType.DMA] * (2 * _NB)
            + [pltpu.VMEM_SHARED((nrows, d), jnp.int32)]
        ),
    )
    def gather_kernel(td_hbm, ts_hbm, id_hbm, is_hbm, gd_hbm, gs_hbm,
                      idx_all, b0, b1, g0, g1, w0, w1, tstage):
        cid = lax.axis_index("c")
        sid = lax.axis_index("s")
        bufs = (b0, b1)
        gsem = (g0, g1)
        wsem = (w0, w1)
        base = sid * per_w

        def run(table, idh, outh):
            pltpu.sync_copy(table.at[pl.ds(sid * rs, rs)],
                            tstage.at[pl.ds(sid * rs, rs)])
            pltpu.sync_copy(idh.at[pl.ds(base, per_w)], idx_all)
            plsc.subcore_barrier()

            def grp(g, carry):
                for b in range(_NB):
                    j = g * _NB + b

                    @pl.when(g > 0)
                    def _():
                        pltpu.make_async_copy(
                            outh.at[pl.ds(0, _CH)], bufs[b], wsem[b]).wait()

                    pltpu.async_copy(
                        tstage.at[idx_all.at[pl.ds(j * _CH, _CH)]],
                        bufs[b], gsem[b])
                for b in range(_NB):
                    j = g * _NB + b
                    pltpu.make_async_copy(
                        outh.at[pl.ds(0, _CH)], bufs[b], gsem[b]).wait()
                    pltpu.async_copy(
                        bufs[b], outh.at[pl.ds(base + j * _CH, _CH)], wsem[b])
                return carry

            lax.fori_loop(0, ngr, grp, 0)
            for b in range(_NB):
                pltpu.make_async_copy(
                    outh.at[pl.ds(0, _CH)], bufs[b], wsem[b]).wait()

        @pl.when(cid == 0)
        def _():
            run(td_hbm, id_hbm, gd_hbm)

        @pl.when(cid == 1)
        def _():
            run(ts_hbm, is_hbm, gs_hbm)

    return gather_kernel(table_d, table_s, idx_d, idx_s)


def _scatter_add(msg, idx, zeros, n):
    ep, d = msg.shape
    npad = zeros.shape[0]
    per_w = ep // _NW
    nch = per_w // _CHS
    ngr = nch // _NBS
    rpt = npad // _NS
    mesh = plsc.VectorSubcoreMesh(core_axis_name="c", subcore_axis_name="s")

    @functools.partial(
        pl.kernel,
        out_type=jax.ShapeDtypeStruct((2, npad, d), _F32),
        mesh=mesh,
        scratch_types=(
            [pltpu.VMEM((nch, _CHS), jnp.int32)]
            + [pltpu.VMEM((_CHS, d), _F32)] * _NBS
            + [pltpu.SemaphoreType.DMA] * (2 * _NBS)
            + [pltpu.VMEM_SHARED((npad, d), _F32)]
        ),
    )
    def scatter_kernel(msg_hbm, idx_hbm, z_hbm, out_hbm,
                       idx_all, b0, b1, l0, l1, a0, a1, acc):
        cid = lax.axis_index("c")
        sid = lax.axis_index("s")
        wid = sid * _NC + cid
        bufs = (b0, b1)
        lsem = (l0, l1)
        asem = (a0, a1)

        pltpu.sync_copy(z_hbm.at[pl.ds(sid * rpt, rpt)],
                        acc.at[pl.ds(sid * rpt, rpt)])
        pltpu.sync_copy(idx_hbm.at[wid], idx_all)
        plsc.subcore_barrier()

        def grp(g, carry):
            for b in range(_NBS):
                j = g * _NBS + b

                @pl.when(g > 0)
                def _():
                    pltpu.make_async_copy(
                        msg_hbm.at[pl.ds(0, _CHS)], bufs[b], asem[b]).wait()

                pltpu.async_copy(
                    msg_hbm.at[pl.ds(wid * per_w + j * _CHS, _CHS)],
                    bufs[b], lsem[b])
            for b in range(_NBS):
                j = g * _NBS + b
                pltpu.make_async_copy(
                    msg_hbm.at[pl.ds(0, _CHS)], bufs[b], lsem[b]).wait()
                pltpu.async_copy(bufs[b], acc.at[idx_all.at[j]], asem[b],
                                 add=True)
            return carry

        lax.fori_loop(0, ngr, grp, 0)
        for b in range(_NBS):
            pltpu.make_async_copy(
                msg_hbm.at[pl.ds(0, _CHS)], bufs[b], asem[b]).wait()
        plsc.subcore_barrier()
        pltpu.sync_copy(acc.at[pl.ds(sid * rpt, rpt)],
                        out_hbm.at[cid, pl.ds(sid * rpt, rpt)])

    return scatter_kernel(msg, idx, zeros)


def kernel(x, edge_index, edge_attr,
           l0_ew1, l0_eb1, l0_ew2, l0_eb2,
           l0_mw1, l0_mb1, l0_mw2, l0_mb2,
           l0_nw1, l0_nb1, l0_nw2, l0_nb2,
           l1_ew1, l1_eb1, l1_ew2, l1_eb2,
           l1_mw1, l1_mb1, l1_mw2, l1_mb2,
           l1_nw1, l1_nb1, l1_nw2, l1_nb2):
    n, d = x.shape
    e = edge_index.shape[1]
    ed = edge_attr.shape[1]
    e_real = e + n
    quant = 2 * _NW * _CH
    ep = ((e_real + quant - 1) // quant) * quant
    eph = ep // 2
    pad = ep - e_real
    npad = ((n + 1 + 8 * _NS - 1) // (8 * _NS)) * (8 * _NS)

    loop = jnp.arange(n, dtype=jnp.int32)
    zpad_i = jnp.zeros((pad,), jnp.int32)
    src = jnp.concatenate([edge_index[0].astype(jnp.int32), loop, zpad_i])
    dst_g = jnp.concatenate([edge_index[1].astype(jnp.int32), loop, zpad_i])
    dst_s = jnp.concatenate([edge_index[1].astype(jnp.int32), loop,
                             jnp.full((pad,), n, jnp.int32)])
    ea = jnp.concatenate(
        [edge_attr, jnp.zeros((n + pad, ed), edge_attr.dtype)], axis=0)
    zeros_acc = jnp.zeros((npad, d), _F32)

    srcs = [src[:eph], src[eph:]]
    dstg = [dst_g[:eph], dst_g[eph:]]
    dsts = [dst_s[:eph].reshape(_NW, -1, _CHS),
            dst_s[eph:].reshape(_NW, -1, _CHS)]
    eas = [ea[:eph], ea[eph:]]

    layers = [
        (l0_ew1, l0_eb1, l0_ew2, l0_eb2, l0_mw1, l0_mb1, l0_mw2, l0_mb2,
         l0_nw1, l0_nb1, l0_nw2, l0_nb2),
        (l1_ew1, l1_eb1, l1_ew2, l1_eb2, l1_mw1, l1_mb1, l1_mw2, l1_mb2,
         l1_nw1, l1_nb1, l1_nw2, l1_nb2),
    ]

    bn = 1000
    be = 2048

    for li, (ew1, eb1, ew2, eb2, mw1, mb1, mw2, mb2,
             nw1, nb1, nw2, nb2) in enumerate(layers):
        wd = jnp.concatenate([ew1[:d], mw1[:d]], axis=1)
        wsc = jnp.concatenate([ew1[d:2 * d], mw1[d:2 * d]], axis=1)
        ne_dtype = jnp.bfloat16 if li == 0 else _F32
        td, ts = _project(x, wd, wsc, bn, npad)
        nes = []
        parts = []
        for h in (0, 1):
            gd, gs = _gather_two(td, ts, dstg[h], srcs[h])
            ne, msg = _edge_mlps(
                gd, gs, eas[h],
                ew1[2 * d:], eb1.reshape(1, d), ew2, eb2.reshape(1, d),
                mw1[2 * d:], mb1.reshape(1, d), mw2, mb2.reshape(1, d), be,
                ne_dtype)
            parts.append(_scatter_add(msg, dsts[h], zeros_acc, n))
            nes.append(ne)
        x = _node_mlp(x, parts[0], parts[1], nw1[:d], nw1[d:],
                      nb1.reshape(1, d), nw2, nb2.reshape(1, d), bn)
        eas = nes

    return (x, jnp.concatenate(eas)[:e_real])


# speedup vs baseline: 1.1625x; 1.0224x over previous
import functools

import jax
import jax.numpy as jnp
from jax import lax
from jax.experimental import pallas as pl
from jax.experimental.pallas import tpu as pltpu
from jax.experimental.pallas import tpu_sc as plsc

_NC = 2
_NS = 16
_NW = _NC * _NS
_CH = 128

_F32 = jnp.float32


def _pack2(t, d):
    hi = lax.bitcast_convert_type(t[:, :d], jnp.int32)
    lo = lax.bitcast_convert_type(t[:, d:], jnp.int32)
    hi = (hi + 0x8000) & jnp.int32(-65536)
    lo = lax.shift_right_logical(lo + 0x8000, 16)
    return hi | lo


def _unpack_hi(g):
    return lax.bitcast_convert_type(g & jnp.int32(-65536), _F32)


def _unpack_lo(g):
    return lax.bitcast_convert_type(lax.shift_left(g, 16), _F32)


def _proj_body(x_ref, wd_ref, ws_ref, od_ref, os_ref):
    xb = x_ref[...]
    d = x_ref.shape[1]
    od_ref[...] = _pack2(
        jnp.dot(xb, wd_ref[...], preferred_element_type=_F32), d)
    os_ref[...] = _pack2(
        jnp.dot(xb, ws_ref[...], preferred_element_type=_F32), d)


def _project(x, wd, ws, bn, nrows):
    n, d = x.shape
    dd = wd.shape[1]
    return pl.pallas_call(
        _proj_body,
        grid=(n // bn,),
        in_specs=[
            pl.BlockSpec((bn, d), lambda i: (i, 0)),
            pl.BlockSpec((d, dd), lambda i: (0, 0)),
            pl.BlockSpec((d, dd), lambda i: (0, 0)),
        ],
        out_specs=[
            pl.BlockSpec((bn, d), lambda i: (i, 0)),
            pl.BlockSpec((bn, d), lambda i: (i, 0)),
        ],
        out_shape=[jax.ShapeDtypeStruct((nrows, d), jnp.int32)] * 2,
    )(x, wd, ws)


def _bdot(a, w):
    return jnp.dot(a.astype(jnp.bfloat16), w.astype(jnp.bfloat16),
                   preferred_element_type=_F32)


def _edge_body(gd_ref, gs_ref, ea_ref, ew1c_ref, eb1_ref, ew2_ref, eb2_ref,
               mw1c_ref, mb1_ref, mw2_ref, mb2_ref, ne_ref, msg_ref):
    gd = gd_ref[...]
    gs = gs_ref[...]
    he = (_unpack_hi(gd) + _unpack_hi(gs) + _bdot(ea_ref[...], ew1c_ref[...])
          + eb1_ref[...])
    he = jnp.maximum(he, 0.0)
    ne = _bdot(he, ew2_ref[...]) + eb2_ref[...]
    hm = (_unpack_lo(gd) + _unpack_lo(gs) + _bdot(ne, mw1c_ref[...])
          + mb1_ref[...])
    hm = jnp.maximum(hm, 0.0)
    msg_ref[...] = _bdot(hm, mw2_ref[...]) + mb2_ref[...]
    ne_ref[...] = ne.astype(ne_ref.dtype)


def _edge_mlps(gd, gs, ea, ew1c, eb1, ew2, eb2, mw1c, mb1, mw2, mb2, be,
               ne_dtype):
    ep, d = gd.shape
    ew = ea.shape[1]
    wspec = lambda sh: pl.BlockSpec(sh, lambda i: (0, 0))
    return pl.pallas_call(
        _edge_body,
        grid=(ep // be,),
        in_specs=[
            pl.BlockSpec((be, d), lambda i: (i, 0)),
            pl.BlockSpec((be, d), lambda i: (i, 0)),
            pl.BlockSpec((be, ew), lambda i: (i, 0)),
            wspec((ew, d)), wspec((1, d)), wspec((d, d)), wspec((1, d)),
            wspec((d, d)), wspec((1, d)), wspec((d, d)), wspec((1, d)),
        ],
        out_specs=[
            pl.BlockSpec((be, d), lambda i: (i, 0)),
            pl.BlockSpec((be, d), lambda i: (i, 0)),
        ],
        out_shape=[jax.ShapeDtypeStruct((ep, d), ne_dtype),
                   jax.ShapeDtypeStruct((ep, d), _F32)],
    )(gd, gs, ea, ew1c, eb1, ew2, eb2, mw1c, mb1, mw2, mb2)


def _node_body(x_ref, p0_ref, p1_ref, p2_ref, p3_ref, w1a_ref, w1b_ref,
               b1_ref, w2_ref, b2_ref, o_ref):
    aggr = p0_ref[0] + p1_ref[0] + p2_ref[0] + p3_ref[0]
    h = (jnp.dot(x_ref[...], w1a_ref[...], preferred_element_type=_F32)
         + jnp.dot(aggr, w1b_ref[...], preferred_element_type=_F32)
         + b1_ref[...])
    h = jnp.maximum(h, 0.0)
    o_ref[...] = jnp.dot(h, w2_ref[...], preferred_element_type=_F32) + b2_ref[...]


def _node_mlp(x, parts_a, parts_b, w1a, w1b, b1, w2, b2, bn):
    n, d = x.shape
    wspec = lambda sh: pl.BlockSpec(sh, lambda i: (0, 0))
    pspec = lambda c: pl.BlockSpec((1, bn, d), lambda i, c=c: (c, i, 0))
    return pl.pallas_call(
        _node_body,
        grid=(n // bn,),
        in_specs=[
            pl.BlockSpec((bn, d), lambda i: (i, 0)),
            pspec(0), pspec(1), pspec(0), pspec(1),
            wspec((d, d)), wspec((d, d)), wspec((1, d)), wspec((d, d)),
            wspec((1, d)),
        ],
        out_specs=pl.BlockSpec((bn, d), lambda i: (i, 0)),
        out_shape=jax.ShapeDtypeStruct((n, d), _F32),
    )(x, parts_a, parts_a, parts_b, parts_b, w1a, w1b, b1, w2, b2)


_NB = 2
_NBS = 2
_CHS = 96


def _gather_two(table_d, table_s, idx_d, idx_s):
    ep = idx_d.shape[0]
    nrows, d = table_d.shape
    per_w = ep // _NS
    nch = per_w // _CH
    ngr = nch // _NB
    rs = nrows // _NS
    mesh = plsc.VectorSubcoreMesh(core_axis_name="c", subcore_axis_name="s")

    @functools.partial(
        pl.kernel,
        out_type=[jax.ShapeDtypeStruct((ep, d), jnp.int32)] * 2,
        mesh=mesh,
        scratch_types=(
            [pltpu.VMEM((per_w,), jnp.int32)]
            + [pltpu.VMEM((_CH, d), jnp.int32)] * _NB
            + [pltpu.SemaphoreType.DMA] * (2 * _NB)
            + [pltpu.VMEM_SHARED((nrows, d), jnp.int32)]
        ),
    )
    def gather_kernel(td_hbm, ts_hbm, id_hbm, is_hbm, gd_hbm, gs_hbm,
                      idx_all, b0, b1, g0, g1, w0, w1, tstage):
        cid = lax.axis_index("c")
        sid = lax.axis_index("s")
        bufs = (b0, b1)
        gsem = (g0, g1)
        wsem = (w0, w1)
        base = sid * per_w

        def run(table, idh, outh):
            pltpu.sync_copy(table.at[pl.ds(sid * rs, rs)],
                            tstage.at[pl.ds(sid * rs, rs)])
            pltpu.sync_copy(idh.at[pl.ds(base, per_w)], idx_all)
            plsc.subcore_barrier()

            def grp(g, carry):
                for b in range(_NB):
                    j = g * _NB + b

                    @pl.when(g > 0)
                    def _():
                        pltpu.make_async_copy(
                            outh.at[pl.ds(0, _CH)], bufs[b], wsem[b]).wait()

                    pltpu.async_copy(
                        tstage.at[idx_all.at[pl.ds(j * _CH, _CH)]],
                        bufs[b], gsem[b])
                for b in range(_NB):
                    j = g * _NB + b
                    pltpu.make_async_copy(
                        outh.at[pl.ds(0, _CH)], bufs[b], gsem[b]).wait()
                    pltpu.async_copy(
                        bufs[b], outh.at[pl.ds(base + j * _CH, _CH)], wsem[b])
                return carry

            lax.fori_loop(0, ngr, grp, 0)
            for b in range(_NB):
                pltpu.make_async_copy(
                    outh.at[pl.ds(0, _CH)], bufs[b], wsem[b]).wait()

        @pl.when(cid == 0)
        def _():
            run(td_hbm, id_hbm, gd_hbm)

        @pl.when(cid == 1)
        def _():
            run(ts_hbm, is_hbm, gs_hbm)

    return gather_kernel(table_d, table_s, idx_d, idx_s)


def _scatter_add(msg, idx, zeros, n):
    ep, d = msg.shape
    npad = zeros.shape[0]
    per_w = ep // _NW
    nch = per_w // _CHS
    ngr = nch // _NBS
    rpt = npad // _NS
    mesh = plsc.VectorSubcoreMesh(core_axis_name="c", subcore_axis_name="s")

    @functools.partial(
        pl.kernel,
        out_type=jax.ShapeDtypeStruct((2, npad, d), _F32),
        mesh=mesh,
        scratch_types=(
            [pltpu.VMEM((nch, _CHS), jnp.int32)]
            + [pltpu.VMEM((_CHS, d), _F32)] * _NBS
            + [pltpu.SemaphoreType.DMA] * (2 * _NBS)
            + [pltpu.VMEM_SHARED((npad, d), _F32)]
        ),
    )
    def scatter_kernel(msg_hbm, idx_hbm, z_hbm, out_hbm,
                       idx_all, b0, b1, l0, l1, a0, a1, acc):
        cid = lax.axis_index("c")
        sid = lax.axis_index("s")
        wid = sid * _NC + cid
        bufs = (b0, b1)
        lsem = (l0, l1)
        asem = (a0, a1)

        pltpu.sync_copy(z_hbm.at[pl.ds(sid * rpt, rpt)],
                        acc.at[pl.ds(sid * rpt, rpt)])
        pltpu.sync_copy(idx_hbm.at[wid], idx_all)
        plsc.subcore_barrier()

        def grp(g, carry):
            for b in range(_NBS):
                j = g * _NBS + b

                @pl.when(g > 0)
                def _():
                    pltpu.make_async_copy(
                        msg_hbm.at[pl.ds(0, _CHS)], bufs[b], asem[b]).wait()

                pltpu.async_copy(
                    msg_hbm.at[pl.ds(wid * per_w + j * _CHS, _CHS)],
                    bufs[b], lsem[b])
            for b in range(_NBS):
                j = g * _NBS + b
                pltpu.make_async_copy(
                    msg_hbm.at[pl.ds(0, _CHS)], bufs[b], lsem[b]).wait()
                pltpu.async_copy(bufs[b], acc.at[idx_all.at[j]], asem[b],
                                 add=True)
            return carry

        lax.fori_loop(0, ngr, grp, 0)
        for b in range(_NBS):
            pltpu.make_async_copy(
                msg_hbm.at[pl.ds(0, _CHS)], bufs[b], asem[b]).wait()
        plsc.subcore_barrier()
        pltpu.sync_copy(acc.at[pl.ds(sid * rpt, rpt)],
                        out_hbm.at[cid, pl.ds(sid * rpt, rpt)])

    return scatter_kernel(msg, idx, zeros)


def kernel(x, edge_index, edge_attr,
           l0_ew1, l0_eb1, l0_ew2, l0_eb2,
           l0_mw1, l0_mb1, l0_mw2, l0_mb2,
           l0_nw1, l0_nb1, l0_nw2, l0_nb2,
           l1_ew1, l1_eb1, l1_ew2, l1_eb2,
           l1_mw1, l1_mb1, l1_mw2, l1_mb2,
           l1_nw1, l1_nb1, l1_nw2, l1_nb2):
    n, d = x.shape
    e = edge_index.shape[1]
    ed = edge_attr.shape[1]
    e_real = e + n
    quant = 2 * _NW * _CH
    ep = ((e_real + quant - 1) // quant) * quant
    eph = ep // 2
    pad = ep - e_real
    npad = ((n + 1 + 8 * _NS - 1) // (8 * _NS)) * (8 * _NS)

    loop = jnp.arange(n, dtype=jnp.int32)
    zpad_i = jnp.zeros((pad,), jnp.int32)
    src = jnp.concatenate([edge_index[0].astype(jnp.int32), loop, zpad_i])
    dst_g = jnp.concatenate([edge_index[1].astype(jnp.int32), loop, zpad_i])
    dst_s = jnp.concatenate([edge_index[1].astype(jnp.int32), loop,
                             jnp.full((pad,), n, jnp.int32)])
    ea = jnp.concatenate(
        [edge_attr, jnp.zeros((n + pad, ed), edge_attr.dtype)], axis=0)
    zeros_acc = jnp.zeros((npad, d), _F32)

    srcs = [src[:eph], src[eph:]]
    dstg = [dst_g[:eph], dst_g[eph:]]
    dsts = [dst_s[:eph].reshape(_NW, -1, _CHS),
            dst_s[eph:].reshape(_NW, -1, _CHS)]
    eas = [ea[:eph], ea[eph:]]

    layers = [
        (l0_ew1, l0_eb1, l0_ew2, l0_eb2, l0_mw1, l0_mb1, l0_mw2, l0_mb2,
         l0_nw1, l0_nb1, l0_nw2, l0_nb2),
        (l1_ew1, l1_eb1, l1_ew2, l1_eb2, l1_mw1, l1_mb1, l1_mw2, l1_mb2,
         l1_nw1, l1_nb1, l1_nw2, l1_nb2),
    ]

    bn = 1000
    be = 4096

    for li, (ew1, eb1, ew2, eb2, mw1, mb1, mw2, mb2,
             nw1, nb1, nw2, nb2) in enumerate(layers):
        wd = jnp.concatenate([ew1[:d], mw1[:d]], axis=1)
        wsc = jnp.concatenate([ew1[d:2 * d], mw1[d:2 * d]], axis=1)
        ne_dtype = jnp.bfloat16 if li == 0 else _F32
        td, ts = _project(x, wd, wsc, bn, npad)
        nes = []
        parts = []
        for h in (0, 1):
            gd, gs = _gather_two(td, ts, dstg[h], srcs[h])
            ne, msg = _edge_mlps(
                gd, gs, eas[h],
                ew1[2 * d:], eb1.reshape(1, d), ew2, eb2.reshape(1, d),
                mw1[2 * d:], mb1.reshape(1, d), mw2, mb2.reshape(1, d), be,
                ne_dtype)
            parts.append(_scatter_add(msg, dsts[h], zeros_acc, n))
            nes.append(ne)
        x = _node_mlp(x, parts[0], parts[1], nw1[:d], nw1[d:],
                      nb1.reshape(1, d), nw2, nb2.reshape(1, d), bn)
        eas = nes

    return (x, jnp.concatenate(eas)[:e_real])
